# baseline (speedup 1.0000x reference)
import functools

import jax
import jax.numpy as jnp
from jax import lax
from jax.experimental import pallas as pl
from jax.experimental.pallas import tpu as pltpu
from jax.experimental.pallas import tpu_sc as plsc

N = 10000
D = 128
WP = 144
NC = 2
NS = 16
NW = NC * NS
L = 16


def _pre_body(h_ref, w_ref, as_ref, ad_ref, xwp_ref, aux_ref):
    xw = jnp.dot(h_ref[...], w_ref[...], preferred_element_type=jnp.float32)
    asr = jnp.sum(xw * as_ref[0:1, :], axis=1, keepdims=True)
    adr = jnp.sum(xw * ad_ref[0:1, :], axis=1, keepdims=True)
    r = xw.shape[0]
    xwp_ref[:, 0:D] = xw
    xwp_ref[:, D:D + 1] = jnp.ones((r, 1), jnp.float32)
    xwp_ref[:, D + 1:WP] = jnp.zeros((r, WP - D - 1), jnp.float32)
    aux_ref[:, 0:1] = asr
    aux_ref[:, 1:2] = adr
    aux_ref[:, 2:16] = jnp.zeros((r, 14), jnp.float32)


def _tc_pre(h, w, asv, adv):
    r = 2000
    grid = (N // r,)
    return pl.pallas_call(
        _pre_body,
        grid=grid,
        in_specs=[
            pl.BlockSpec((r, D), lambda i: (i, 0)),
            pl.BlockSpec((D, D), lambda i: (0, 0)),
            pl.BlockSpec((8, D), lambda i: (0, 0)),
            pl.BlockSpec((8, D), lambda i: (0, 0)),
        ],
        out_specs=[
            pl.BlockSpec((r, WP), lambda i: (i, 0)),
            pl.BlockSpec((r, 16), lambda i: (i, 0)),
        ],
        out_shape=[
            jax.ShapeDtypeStruct((N, WP), jnp.float32),
            jax.ShapeDtypeStruct((N, 16), jnp.float32),
        ],
    )(h, w, asv, adv)


def _post_body(a0_ref, a1_ref, xwp_ref, aux_ref, b_ref, o_ref, *, relu):
    el_s = aux_ref[:, 0:1] + aux_ref[:, 1:2]
    el = jnp.exp(jnp.maximum(el_s, 0.2 * el_s))
    num = a0_ref[:, 0:D] + a1_ref[:, 0:D] + el * xwp_ref[:, 0:D]
    den = a0_ref[:, D:D + 1] + a1_ref[:, D:D + 1] + el + 1e-16
    h = num / den + b_ref[0:1, :]
    if relu:
        h = jnp.maximum(h, 0.0)
    o_ref[...] = h


def _tc_post(a0, a1, xwp, aux, bv, relu):
    r = 2000
    grid = (N // r,)
    return pl.pallas_call(
        functools.partial(_post_body, relu=relu),
        grid=grid,
        in_specs=[
            pl.BlockSpec((r, WP), lambda i: (i, 0)),
            pl.BlockSpec((r, WP), lambda i: (i, 0)),
            pl.BlockSpec((r, WP), lambda i: (i, 0)),
            pl.BlockSpec((r, 16), lambda i: (i, 0)),
            pl.BlockSpec((8, D), lambda i: (0, 0)),
        ],
        out_specs=pl.BlockSpec((r, D), lambda i: (i, 0)),
        out_shape=jax.ShapeDtypeStruct((N, D), jnp.float32),
    )(a0, a1, xwp, aux, bv)


@functools.lru_cache(maxsize=None)
def _mesh():
    return plsc.VectorSubcoreMesh(
        core_axis_name="c", subcore_axis_name="s",
        num_cores=NC, num_subcores=NS)


def _sc_edge(xwp, aux, src_idx, dst_idx):
    e = src_idx.shape[0]
    ept = e // NW
    chunk = 80
    nchunk = ept // chunk
    ngrp = chunk // L
    rows_per_tile = N // NS

    @functools.partial(
        pl.kernel,
        out_type=jax.ShapeDtypeStruct((NC, N, WP), jnp.float32),
        mesh=_mesh(),
        compiler_params=pltpu.CompilerParams(
            use_tc_tiling_on_sc=False, needs_layout_passes=False),
        scratch_types=[
            pltpu.VMEM((2, chunk), jnp.int32),
            pltpu.VMEM((2, chunk), jnp.int32),
            pltpu.VMEM((2, chunk, 16), jnp.float32),
            pltpu.VMEM((2, chunk, 16), jnp.float32),
            pltpu.VMEM((2, chunk, WP), jnp.float32),
            pltpu.VMEM_SHARED((N, WP), jnp.float32),
            pltpu.SemaphoreType.DMA,
            pltpu.SemaphoreType.DMA,
        ],
    )
    def edge_kernel(xwp_hbm, aux_hbm, si_hbm, di_hbm, out_hbm,
                    srcc_v, dstc_v, auxs_v, auxd_v, rows_v, acc_sh,
                    gsem, ssem):
        cid = lax.axis_index("c")
        sid = lax.axis_index("s")
        wid = cid * NS + sid
        eb = wid * ept

        z16 = jnp.zeros((L,), jnp.float32)

        @pl.loop(0, chunk)
        def _(r):
            for q in range(WP // L):
                rows_v[0, r, pl.ds(q * L, L)] = z16

        row0 = sid * rows_per_tile
        nfull, rem = rows_per_tile // chunk, rows_per_tile % chunk
        for i in range(nfull):
            pltpu.sync_copy(rows_v.at[0],
                            acc_sh.at[pl.ds(row0 + i * chunk, chunk)])
        if rem:
            pltpu.sync_copy(rows_v.at[0, pl.ds(0, rem)],
                            acc_sh.at[pl.ds(row0 + nfull * chunk, rem)])
        plsc.subcore_barrier()

        zero16 = jnp.zeros((L,), jnp.int32)
        one16 = jnp.ones((L,), jnp.int32)
        iota16 = jax.lax.iota(jnp.int32, L)

        def stage_and_gather(c, s):
            base = eb + c * chunk
            pltpu.sync_copy(si_hbm.at[pl.ds(base, chunk)], srcc_v.at[s])
            pltpu.sync_copy(di_hbm.at[pl.ds(base, chunk)], dstc_v.at[s])
            pltpu.async_copy(xwp_hbm.at[srcc_v.at[s]], rows_v.at[s], gsem)
            pltpu.async_copy(aux_hbm.at[srcc_v.at[s]], auxs_v.at[s], gsem)
            pltpu.async_copy(aux_hbm.at[dstc_v.at[s]], auxd_v.at[s], gsem)

        def wait_gathers(s):
            pltpu.make_async_copy(
                xwp_hbm.at[srcc_v.at[s]], rows_v.at[s], gsem).wait()
            pltpu.make_async_copy(
                aux_hbm.at[srcc_v.at[s]], auxs_v.at[s], gsem).wait()
            pltpu.make_async_copy(
                aux_hbm.at[srcc_v.at[s]], auxd_v.at[s], gsem).wait()

        def wait_scatter(s):
            pltpu.make_async_copy(
                rows_v.at[s], acc_sh.at[dstc_v.at[s]], ssem).wait()

        stage_and_gather(0, 0)

        @pl.loop(0, nchunk)
        def _(c):
            b = lax.rem(c, 2)
            nb = 1 - b

            @pl.when(c + 1 < nchunk)
            def _():
                @pl.when(c >= 1)
                def _():
                    wait_scatter(nb)
                stage_and_gather(c + 1, nb)

            wait_gathers(b)

            @pl.loop(0, ngrp)
            def _(g):
                idx16 = iota16 + g * L
                a_s = plsc.load_gather(auxs_v, [jnp.full((L,), b, jnp.int32),
                                                idx16, zero16])
                a_d = plsc.load_gather(auxd_v, [jnp.full((L,), b, jnp.int32),
                                                idx16, one16])
                ez = a_s + a_d
                ez = jnp.maximum(ez, 0.2 * ez)
                ee = jnp.exp(ez)
                for j in range(L):
                    ev = lax.gather(
                        ee, jnp.full((L, 1), j, jnp.int32),
                        lax.GatherDimensionNumbers(
                            offset_dims=(), collapsed_slice_dims=(0,),
                            start_index_map=(0,)),
                        (1,), mode=lax.GatherScatterMode.PROMISE_IN_BOUNDS)
                    r = g * L + j
                    for q in range(WP // L):
                        sl = pl.ds(q * L, L)
                        rows_v[b, r, sl] = rows_v[b, r, sl] * ev

            pltpu.async_copy(rows_v.at[b], acc_sh.at[dstc_v.at[b]], ssem,
                             add=True)

        wait_scatter(0)
        wait_scatter(1)

        plsc.subcore_barrier()
        pltpu.sync_copy(acc_sh.at[pl.ds(row0, rows_per_tile)],
                        out_hbm.at[cid, pl.ds(row0, rows_per_tile)])

    return edge_kernel(xwp, aux, src_idx, dst_idx)


def _row8(v):
    return jnp.zeros((8, D), jnp.float32).at[0].set(v.reshape(-1))


def kernel(x, edge_index, W1, a_src1, a_dst1, b1, W2, a_src2, a_dst2, b2):
    asv1, adv1 = _row8(a_src1), _row8(a_dst1)
    asv2, adv2 = _row8(a_src2), _row8(a_dst2)
    bv1, bv2 = _row8(b1), _row8(b2)

    src_idx, dst_idx = edge_index[0], edge_index[1]

    xwp1, aux1 = _tc_pre(x, W1, asv1, adv1)
    acc = _sc_edge(xwp1, aux1, src_idx, dst_idx)
    h = _tc_post(acc[0], acc[1], xwp1, aux1, bv1, True)

    xwp2, aux2 = _tc_pre(h, W2, asv2, adv2)
    acc2 = _sc_edge(xwp2, aux2, src_idx, dst_idx)
    return _tc_post(acc2[0], acc2[1], xwp2, aux2, bv2, False)


# speedup vs baseline: 2.3623x; 2.3623x over previous
import functools

import jax
import jax.numpy as jnp
from jax import lax
from jax.experimental import pallas as pl
from jax.experimental.pallas import tpu as pltpu
from jax.experimental.pallas import tpu_sc as plsc

N = 10000
D = 128
WP = 144
NC = 2
NS = 16
NW = NC * NS
L = 16


def _pre_body(h_ref, w_ref, as_ref, ad_ref, xwp_ref, aux_ref):
    xw = jnp.dot(h_ref[...], w_ref[...], preferred_element_type=jnp.float32)
    asr = jnp.sum(xw * as_ref[0:1, :], axis=1, keepdims=True)
    adr = jnp.sum(xw * ad_ref[0:1, :], axis=1, keepdims=True)
    r = xw.shape[0]
    xwp_ref[:, 0:D] = xw
    xwp_ref[:, D:D + 1] = jnp.ones((r, 1), jnp.float32)
    xwp_ref[:, D + 1:D + 2] = asr
    xwp_ref[:, D + 2:WP] = jnp.zeros((r, WP - D - 2), jnp.float32)
    aux_ref[:, 0:1] = asr
    aux_ref[:, 1:2] = adr
    aux_ref[:, 2:16] = jnp.zeros((r, 14), jnp.float32)


def _tc_pre(h, w, asv, adv):
    r = 2000
    grid = (N // r,)
    return pl.pallas_call(
        _pre_body,
        grid=grid,
        in_specs=[
            pl.BlockSpec((r, D), lambda i: (i, 0)),
            pl.BlockSpec((D, D), lambda i: (0, 0)),
            pl.BlockSpec((8, D), lambda i: (0, 0)),
            pl.BlockSpec((8, D), lambda i: (0, 0)),
        ],
        out_specs=[
            pl.BlockSpec((r, WP), lambda i: (i, 0)),
            pl.BlockSpec((r, 16), lambda i: (i, 0)),
        ],
        out_shape=[
            jax.ShapeDtypeStruct((N, WP), jnp.float32),
            jax.ShapeDtypeStruct((N, 16), jnp.float32),
        ],
    )(h, w, asv, adv)


def _post_body(a0_ref, a1_ref, xwp_ref, aux_ref, b_ref, o_ref, *, relu):
    el_s = aux_ref[:, 0:1] + aux_ref[:, 1:2]
    el = jnp.exp(jnp.maximum(el_s, 0.2 * el_s))
    num = a0_ref[:, 0:D] + a1_ref[:, 0:D] + el * xwp_ref[:, 0:D]
    den = a0_ref[:, D:D + 1] + a1_ref[:, D:D + 1] + el + 1e-16
    h = num / den + b_ref[0:1, :]
    if relu:
        h = jnp.maximum(h, 0.0)
    o_ref[...] = h


def _tc_post(a0, a1, xwp, aux, bv, relu):
    r = 2000
    grid = (N // r,)
    return pl.pallas_call(
        functools.partial(_post_body, relu=relu),
        grid=grid,
        in_specs=[
            pl.BlockSpec((r, WP), lambda i: (i, 0)),
            pl.BlockSpec((r, WP), lambda i: (i, 0)),
            pl.BlockSpec((r, WP), lambda i: (i, 0)),
            pl.BlockSpec((r, 16), lambda i: (i, 0)),
            pl.BlockSpec((8, D), lambda i: (0, 0)),
        ],
        out_specs=pl.BlockSpec((r, D), lambda i: (i, 0)),
        out_shape=jax.ShapeDtypeStruct((N, D), jnp.float32),
    )(a0, a1, xwp, aux, bv)


@functools.lru_cache(maxsize=None)
def _mesh():
    return plsc.VectorSubcoreMesh(
        core_axis_name="c", subcore_axis_name="s",
        num_cores=NC, num_subcores=NS)


def _sc_edge(xwp, aux, src_idx, dst_idx):
    e = src_idx.shape[0]
    ept = e // NW
    chunk = 80
    nchunk = ept // chunk
    ngrp = chunk // L
    rows_per_tile = N // NS

    @functools.partial(
        pl.kernel,
        out_type=jax.ShapeDtypeStruct((NC, N, WP), jnp.float32),
        mesh=_mesh(),
        compiler_params=pltpu.CompilerParams(
            use_tc_tiling_on_sc=False, needs_layout_passes=False),
        scratch_types=[
            pltpu.VMEM((3, chunk), jnp.int32),
            pltpu.VMEM((3, chunk), jnp.int32),
            pltpu.VMEM((3, chunk, 16), jnp.float32),
            pltpu.VMEM((3, chunk, WP), jnp.float32),
            pltpu.VMEM_SHARED((N, WP), jnp.float32),
            pltpu.SemaphoreType.DMA,
            pltpu.SemaphoreType.DMA,
        ],
    )
    def edge_kernel(xwp_hbm, aux_hbm, si_hbm, di_hbm, out_hbm,
                    srcc_v, dstc_v, auxd_v, rows_v, acc_sh,
                    gsem, ssem):
        cid = lax.axis_index("c")
        sid = lax.axis_index("s")
        wid = cid * NS + sid
        eb = wid * ept

        z16 = jnp.zeros((L,), jnp.float32)

        @pl.loop(0, chunk)
        def _(r):
            for q in range(WP // L):
                rows_v[0, r, pl.ds(q * L, L)] = z16

        row0 = sid * rows_per_tile
        nfull, rem = rows_per_tile // chunk, rows_per_tile % chunk
        for i in range(nfull):
            pltpu.sync_copy(rows_v.at[0],
                            acc_sh.at[pl.ds(row0 + i * chunk, chunk)])
        if rem:
            pltpu.sync_copy(rows_v.at[0, pl.ds(0, rem)],
                            acc_sh.at[pl.ds(row0 + nfull * chunk, rem)])
        plsc.subcore_barrier()

        one16 = jnp.ones((L,), jnp.int32)
        iota16 = jax.lax.iota(jnp.int32, L)

        def stage_and_gather(c, s):
            base = eb + c * chunk
            pltpu.sync_copy(si_hbm.at[pl.ds(base, chunk)], srcc_v.at[s])
            pltpu.sync_copy(di_hbm.at[pl.ds(base, chunk)], dstc_v.at[s])
            pltpu.async_copy(xwp_hbm.at[srcc_v.at[s]], rows_v.at[s], gsem)
            pltpu.async_copy(aux_hbm.at[dstc_v.at[s]], auxd_v.at[s], gsem)

        def wait_gathers(s):
            pltpu.make_async_copy(
                xwp_hbm.at[srcc_v.at[s]], rows_v.at[s], gsem).wait()
            pltpu.make_async_copy(
                aux_hbm.at[srcc_v.at[s]], auxd_v.at[s], gsem).wait()

        def wait_scatter(s):
            pltpu.make_async_copy(
                rows_v.at[s], acc_sh.at[dstc_v.at[s]], ssem).wait()

        def compute(s):
            sv = jnp.full((L,), s, jnp.int32)

            @pl.loop(0, ngrp)
            def _(g):
                idx16 = iota16 + g * L
                a_s = plsc.load_gather(
                    rows_v, [sv, idx16, jnp.full((L,), D + 1, jnp.int32)])
                a_d = plsc.load_gather(auxd_v, [sv, idx16, one16])
                ez = a_s + a_d
                ez = jnp.maximum(ez, 0.2 * ez)
                ee = jnp.exp(ez)
                for j in range(L):
                    ev = lax.gather(
                        ee, jnp.full((L, 1), j, jnp.int32),
                        lax.GatherDimensionNumbers(
                            offset_dims=(), collapsed_slice_dims=(0,),
                            start_index_map=(0,)),
                        (1,), mode=lax.GatherScatterMode.PROMISE_IN_BOUNDS)
                    r = g * L + j
                    for q in range(D // L):
                        sl = pl.ds(q * L, L)
                        rows_v[s, r, sl] = rows_v[s, r, sl] * ev
                    rows_v[s, r, pl.ds(D, L)] = ev

        def scatter(s):
            pltpu.async_copy(rows_v.at[s], acc_sh.at[dstc_v.at[s]], ssem,
                             add=True)

        def sub_iter(c, s, prefetch, may_be_first):
            wait_gathers(s)
            compute(s)
            sp = (s + 2) % 3
            if prefetch:
                if may_be_first:
                    @pl.when(c >= 1)
                    def _():
                        wait_scatter(sp)
                else:
                    wait_scatter(sp)
                stage_and_gather(c + 2, sp)
            scatter(s)

        stage_and_gather(0, 0)
        stage_and_gather(1, 1)

        @pl.loop(0, nchunk // 3)
        def _(t):
            c0 = t * 3
            sub_iter(c0, 0, True, True)
            sub_iter(c0 + 1, 1, True, False)
            sub_iter(c0 + 2, 2, True, False)

        nmain = (nchunk // 3) * 3
        for k in range(nchunk - nmain):
            sub_iter(nmain + k, k % 3, False, False)

        for s in range(3):
            wait_scatter(s)

        plsc.subcore_barrier()
        pltpu.sync_copy(acc_sh.at[pl.ds(row0, rows_per_tile)],
                        out_hbm.at[cid, pl.ds(row0, rows_per_tile)])

    return edge_kernel(xwp, aux, src_idx, dst_idx)


def _row8(v):
    return jnp.zeros((8, D), jnp.float32).at[0].set(v.reshape(-1))


def kernel(x, edge_index, W1, a_src1, a_dst1, b1, W2, a_src2, a_dst2, b2):
    asv1, adv1 = _row8(a_src1), _row8(a_dst1)
    asv2, adv2 = _row8(a_src2), _row8(a_dst2)
    bv1, bv2 = _row8(b1), _row8(b2)

    src_idx, dst_idx = edge_index[0], edge_index[1]

    xwp1, aux1 = _tc_pre(x, W1, asv1, adv1)
    acc = _sc_edge(xwp1, aux1, src_idx, dst_idx)
    h = _tc_post(acc[0], acc[1], xwp1, aux1, bv1, True)

    xwp2, aux2 = _tc_pre(h, W2, asv2, adv2)
    acc2 = _sc_edge(xwp2, aux2, src_idx, dst_idx)
    return _tc_post(acc2[0], acc2[1], xwp2, aux2, bv2, False)


# speedup vs baseline: 2.4920x; 1.0549x over previous
import functools

import jax
import jax.numpy as jnp
from jax import lax
from jax.experimental import pallas as pl
from jax.experimental.pallas import tpu as pltpu
from jax.experimental.pallas import tpu_sc as plsc

N = 10000
D = 128
WP = 144
NC = 2
NS = 16
NW = NC * NS
L = 16


def _pre_math(h, w_ref, as_ref, ad_ref, xwp_ref, aux_ref):
    xw = jnp.dot(h, w_ref[...], preferred_element_type=jnp.float32)
    asr = jnp.sum(xw * as_ref[0:1, :], axis=1, keepdims=True)
    adr = jnp.sum(xw * ad_ref[0:1, :], axis=1, keepdims=True)
    r = xw.shape[0]
    xwp_ref[:, 0:D] = xw
    xwp_ref[:, D:D + 1] = jnp.ones((r, 1), jnp.float32)
    xwp_ref[:, D + 1:D + 2] = asr
    xwp_ref[:, D + 2:WP] = jnp.zeros((r, WP - D - 2), jnp.float32)
    aux_ref[:, 0:1] = asr
    aux_ref[:, 1:2] = adr
    aux_ref[:, 2:16] = jnp.zeros((r, 14), jnp.float32)


def _pre_body(h_ref, w_ref, as_ref, ad_ref, xwp_ref, aux_ref):
    _pre_math(h_ref[...], w_ref, as_ref, ad_ref, xwp_ref, aux_ref)


def _tc_pre(h, w, asv, adv):
    r = 2000
    grid = (N // r,)
    return pl.pallas_call(
        _pre_body,
        grid=grid,
        in_specs=[
            pl.BlockSpec((r, D), lambda i: (i, 0)),
            pl.BlockSpec((D, D), lambda i: (0, 0)),
            pl.BlockSpec((8, D), lambda i: (0, 0)),
            pl.BlockSpec((8, D), lambda i: (0, 0)),
        ],
        out_specs=[
            pl.BlockSpec((r, WP), lambda i: (i, 0)),
            pl.BlockSpec((r, 16), lambda i: (i, 0)),
        ],
        out_shape=[
            jax.ShapeDtypeStruct((N, WP), jnp.float32),
            jax.ShapeDtypeStruct((N, 16), jnp.float32),
        ],
    )(h, w, asv, adv)


def _post_h(a0_ref, a1_ref, xwp_ref, aux_ref, b_ref, relu):
    el_s = aux_ref[:, 0:1] + aux_ref[:, 1:2]
    el = jnp.exp(jnp.maximum(el_s, 0.2 * el_s))
    num = a0_ref[:, 0:D] + a1_ref[:, 0:D] + el * xwp_ref[:, 0:D]
    den = a0_ref[:, D:D + 1] + a1_ref[:, D:D + 1] + el + 1e-16
    h = num / den + b_ref[0:1, :]
    if relu:
        h = jnp.maximum(h, 0.0)
    return h


def _post_body(a0_ref, a1_ref, xwp_ref, aux_ref, b_ref, o_ref):
    o_ref[...] = _post_h(a0_ref, a1_ref, xwp_ref, aux_ref, b_ref, False)


_ACC_SPECS = [
    pl.BlockSpec((2000, WP), lambda i: (i, 0)),
    pl.BlockSpec((2000, WP), lambda i: (i + N // 2000, 0)),
]


def _tc_post(acc2d, xwp, aux, bv):
    r = 2000
    grid = (N // r,)
    return pl.pallas_call(
        _post_body,
        grid=grid,
        in_specs=_ACC_SPECS + [
            pl.BlockSpec((r, WP), lambda i: (i, 0)),
            pl.BlockSpec((r, 16), lambda i: (i, 0)),
            pl.BlockSpec((8, D), lambda i: (0, 0)),
        ],
        out_specs=pl.BlockSpec((r, D), lambda i: (i, 0)),
        out_shape=jax.ShapeDtypeStruct((N, D), jnp.float32),
    )(acc2d, acc2d, xwp, aux, bv)


def _postpre_body(a0_ref, a1_ref, xwp_ref, aux_ref, b_ref,
                  w_ref, as_ref, ad_ref, xwp2_ref, aux2_ref):
    h = _post_h(a0_ref, a1_ref, xwp_ref, aux_ref, b_ref, True)
    _pre_math(h, w_ref, as_ref, ad_ref, xwp2_ref, aux2_ref)


def _tc_postpre(acc2d, xwp, aux, bv, w, asv, adv):
    r = 2000
    grid = (N // r,)
    return pl.pallas_call(
        _postpre_body,
        grid=grid,
        in_specs=_ACC_SPECS + [
            pl.BlockSpec((r, WP), lambda i: (i, 0)),
            pl.BlockSpec((r, 16), lambda i: (i, 0)),
            pl.BlockSpec((8, D), lambda i: (0, 0)),
            pl.BlockSpec((D, D), lambda i: (0, 0)),
            pl.BlockSpec((8, D), lambda i: (0, 0)),
            pl.BlockSpec((8, D), lambda i: (0, 0)),
        ],
        out_specs=[
            pl.BlockSpec((r, WP), lambda i: (i, 0)),
            pl.BlockSpec((r, 16), lambda i: (i, 0)),
        ],
        out_shape=[
            jax.ShapeDtypeStruct((N, WP), jnp.float32),
            jax.ShapeDtypeStruct((N, 16), jnp.float32),
        ],
    )(acc2d, acc2d, xwp, aux, bv, w, asv, adv)


@functools.lru_cache(maxsize=None)
def _mesh():
    return plsc.VectorSubcoreMesh(
        core_axis_name="c", subcore_axis_name="s",
        num_cores=NC, num_subcores=NS)


def _sc_edge(xwp, aux, src_idx, dst_idx):
    e = src_idx.shape[0]
    ept = e // NW
    chunk = 80
    nchunk = ept // chunk
    ngrp = chunk // L
    rows_per_tile = N // NS

    @functools.partial(
        pl.kernel,
        out_type=jax.ShapeDtypeStruct((NC, N, WP), jnp.float32),
        mesh=_mesh(),
        compiler_params=pltpu.CompilerParams(
            use_tc_tiling_on_sc=False, needs_layout_passes=False),
        scratch_types=[
            pltpu.VMEM((3, chunk), jnp.int32),
            pltpu.VMEM((3, chunk), jnp.int32),
            pltpu.VMEM((3, chunk, 16), jnp.float32),
            pltpu.VMEM((3, chunk, WP), jnp.float32),
            pltpu.VMEM_SHARED((N, WP), jnp.float32),
            pltpu.SemaphoreType.DMA,
            pltpu.SemaphoreType.DMA,
        ],
    )
    def edge_kernel(xwp_hbm, aux_hbm, si_hbm, di_hbm, out_hbm,
                    srcc_v, dstc_v, auxd_v, rows_v, acc_sh,
                    gsem, ssem):
        cid = lax.axis_index("c")
        sid = lax.axis_index("s")
        wid = cid * NS + sid
        eb = wid * ept

        z16 = jnp.zeros((L,), jnp.float32)

        @pl.loop(0, chunk)
        def _(r):
            for q in range(WP // L):
                rows_v[0, r, pl.ds(q * L, L)] = z16

        row0 = sid * rows_per_tile
        nfull, rem = rows_per_tile // chunk, rows_per_tile % chunk
        for i in range(nfull):
            pltpu.sync_copy(rows_v.at[0],
                            acc_sh.at[pl.ds(row0 + i * chunk, chunk)])
        if rem:
            pltpu.sync_copy(rows_v.at[0, pl.ds(0, rem)],
                            acc_sh.at[pl.ds(row0 + nfull * chunk, rem)])
        plsc.subcore_barrier()

        one16 = jnp.ones((L,), jnp.int32)
        iota16 = jax.lax.iota(jnp.int32, L)

        def stage_and_gather(c, s):
            base = eb + c * chunk
            pltpu.sync_copy(si_hbm.at[pl.ds(base, chunk)], srcc_v.at[s])
            pltpu.sync_copy(di_hbm.at[pl.ds(base, chunk)], dstc_v.at[s])
            pltpu.async_copy(xwp_hbm.at[srcc_v.at[s]], rows_v.at[s], gsem)
            pltpu.async_copy(aux_hbm.at[dstc_v.at[s]], auxd_v.at[s], gsem)

        def wait_gathers(s):
            pltpu.make_async_copy(
                xwp_hbm.at[srcc_v.at[s]], rows_v.at[s], gsem).wait()
            pltpu.make_async_copy(
                aux_hbm.at[srcc_v.at[s]], auxd_v.at[s], gsem).wait()

        def wait_scatter(s):
            pltpu.make_async_copy(
                rows_v.at[s], acc_sh.at[dstc_v.at[s]], ssem).wait()

        def compute(s):
            sv = jnp.full((L,), s, jnp.int32)

            @pl.loop(0, ngrp)
            def _(g):
                idx16 = iota16 + g * L
                a_s = plsc.load_gather(
                    rows_v, [sv, idx16, jnp.full((L,), D + 1, jnp.int32)])
                a_d = plsc.load_gather(auxd_v, [sv, idx16, one16])
                ez = a_s + a_d
                ez = jnp.maximum(ez, 0.2 * ez)
                ee = jnp.exp(ez)
                for j in range(L):
                    ev = lax.gather(
                        ee, jnp.full((L, 1), j, jnp.int32),
                        lax.GatherDimensionNumbers(
                            offset_dims=(), collapsed_slice_dims=(0,),
                            start_index_map=(0,)),
                        (1,), mode=lax.GatherScatterMode.PROMISE_IN_BOUNDS)
                    r = g * L + j
                    for q in range(D // L):
                        sl = pl.ds(q * L, L)
                        rows_v[s, r, sl] = rows_v[s, r, sl] * ev
                    rows_v[s, r, pl.ds(D, L)] = ev

        def scatter(s):
            pltpu.async_copy(rows_v.at[s], acc_sh.at[dstc_v.at[s]], ssem,
                             add=True)

        def sub_iter(c, s, prefetch, may_be_first):
            wait_gathers(s)
            compute(s)
            sp = (s + 2) % 3
            if prefetch:
                if may_be_first:
                    @pl.when(c >= 1)
                    def _():
                        wait_scatter(sp)
                else:
                    wait_scatter(sp)
                stage_and_gather(c + 2, sp)
            scatter(s)

        stage_and_gather(0, 0)
        stage_and_gather(1, 1)

        @pl.loop(0, nchunk // 3)
        def _(t):
            c0 = t * 3
            sub_iter(c0, 0, True, True)
            sub_iter(c0 + 1, 1, True, False)
            sub_iter(c0 + 2, 2, True, False)

        nmain = (nchunk // 3) * 3
        for k in range(nchunk - nmain):
            sub_iter(nmain + k, k % 3, False, False)

        for s in range(3):
            wait_scatter(s)

        plsc.subcore_barrier()
        pltpu.sync_copy(acc_sh.at[pl.ds(row0, rows_per_tile)],
                        out_hbm.at[cid, pl.ds(row0, rows_per_tile)])

    return edge_kernel(xwp, aux, src_idx, dst_idx)


def _row8(v):
    return jnp.zeros((8, D), jnp.float32).at[0].set(v.reshape(-1))


def kernel(x, edge_index, W1, a_src1, a_dst1, b1, W2, a_src2, a_dst2, b2):
    asv1, adv1 = _row8(a_src1), _row8(a_dst1)
    asv2, adv2 = _row8(a_src2), _row8(a_dst2)
    bv1, bv2 = _row8(b1), _row8(b2)

    src_idx, dst_idx = edge_index[0], edge_index[1]

    xwp1, aux1 = _tc_pre(x, W1, asv1, adv1)
    acc = _sc_edge(xwp1, aux1, src_idx, dst_idx).reshape(2 * N, WP)
    xwp2, aux2 = _tc_postpre(acc, xwp1, aux1, bv1, W2, asv2, adv2)
    acc2 = _sc_edge(xwp2, aux2, src_idx, dst_idx).reshape(2 * N, WP)
    return _tc_post(acc2, xwp2, aux2, bv2)


# speedup vs baseline: 3.0329x; 1.2171x over previous
import functools

import jax
import jax.numpy as jnp
from jax import lax
from jax.experimental import pallas as pl
from jax.experimental.pallas import tpu as pltpu
from jax.experimental.pallas import tpu_sc as plsc

N = 10000
D = 128
WP = 144
NC = 2
NS = 16
NW = NC * NS
L = 16


def _pre_math(h, w_ref, as_ref, ad_ref, xwp_ref, aux_ref):
    xw = jnp.dot(h, w_ref[...], preferred_element_type=jnp.float32)
    asr = jnp.sum(xw * as_ref[0:1, :], axis=1, keepdims=True)
    adr = jnp.sum(xw * ad_ref[0:1, :], axis=1, keepdims=True)
    r = xw.shape[0]
    xwp_ref[:, 0:D] = xw
    xwp_ref[:, D:D + 1] = jnp.ones((r, 1), jnp.float32)
    xwp_ref[:, D + 1:D + 2] = asr
    xwp_ref[:, D + 2:WP] = jnp.zeros((r, WP - D - 2), jnp.float32)
    aux_ref[:, 0:1] = asr
    aux_ref[:, 1:2] = adr
    aux_ref[:, 2:16] = jnp.zeros((r, 14), jnp.float32)


def _pre_body(h_ref, w_ref, as_ref, ad_ref, xwp_ref, aux_ref):
    _pre_math(h_ref[...], w_ref, as_ref, ad_ref, xwp_ref, aux_ref)


def _tc_pre(h, w, asv, adv):
    r = 2000
    grid = (N // r,)
    return pl.pallas_call(
        _pre_body,
        grid=grid,
        in_specs=[
            pl.BlockSpec((r, D), lambda i: (i, 0)),
            pl.BlockSpec((D, D), lambda i: (0, 0)),
            pl.BlockSpec((8, D), lambda i: (0, 0)),
            pl.BlockSpec((8, D), lambda i: (0, 0)),
        ],
        out_specs=[
            pl.BlockSpec((r, WP), lambda i: (i, 0)),
            pl.BlockSpec((r, 16), lambda i: (i, 0)),
        ],
        out_shape=[
            jax.ShapeDtypeStruct((N, WP), jnp.float32),
            jax.ShapeDtypeStruct((N, 16), jnp.float32),
        ],
    )(h, w, asv, adv)


def _post_h(a0_ref, a1_ref, xwp_ref, aux_ref, b_ref, relu):
    el_s = aux_ref[:, 0:1] + aux_ref[:, 1:2]
    el = jnp.exp(jnp.maximum(el_s, 0.2 * el_s))
    num = a0_ref[:, 0:D] + a1_ref[:, 0:D] + el * xwp_ref[:, 0:D]
    den = a0_ref[:, D:D + 1] + a1_ref[:, D:D + 1] + el + 1e-16
    h = num / den + b_ref[0:1, :]
    if relu:
        h = jnp.maximum(h, 0.0)
    return h


def _post_body(a0_ref, a1_ref, xwp_ref, aux_ref, b_ref, o_ref):
    o_ref[...] = _post_h(a0_ref, a1_ref, xwp_ref, aux_ref, b_ref, False)


_ACC_SPECS = [
    pl.BlockSpec((2000, WP), lambda i: (i, 0)),
    pl.BlockSpec((2000, WP), lambda i: (i + N // 2000, 0)),
]


def _tc_post(acc2d, xwp, aux, bv):
    r = 2000
    grid = (N // r,)
    return pl.pallas_call(
        _post_body,
        grid=grid,
        in_specs=_ACC_SPECS + [
            pl.BlockSpec((r, WP), lambda i: (i, 0)),
            pl.BlockSpec((r, 16), lambda i: (i, 0)),
            pl.BlockSpec((8, D), lambda i: (0, 0)),
        ],
        out_specs=pl.BlockSpec((r, D), lambda i: (i, 0)),
        out_shape=jax.ShapeDtypeStruct((N, D), jnp.float32),
    )(acc2d, acc2d, xwp, aux, bv)


def _postpre_body(a0_ref, a1_ref, xwp_ref, aux_ref, b_ref,
                  w_ref, as_ref, ad_ref, xwp2_ref, aux2_ref):
    h = _post_h(a0_ref, a1_ref, xwp_ref, aux_ref, b_ref, True)
    _pre_math(h, w_ref, as_ref, ad_ref, xwp2_ref, aux2_ref)


def _tc_postpre(acc2d, xwp, aux, bv, w, asv, adv):
    r = 2000
    grid = (N // r,)
    return pl.pallas_call(
        _postpre_body,
        grid=grid,
        in_specs=_ACC_SPECS + [
            pl.BlockSpec((r, WP), lambda i: (i, 0)),
            pl.BlockSpec((r, 16), lambda i: (i, 0)),
            pl.BlockSpec((8, D), lambda i: (0, 0)),
            pl.BlockSpec((D, D), lambda i: (0, 0)),
            pl.BlockSpec((8, D), lambda i: (0, 0)),
            pl.BlockSpec((8, D), lambda i: (0, 0)),
        ],
        out_specs=[
            pl.BlockSpec((r, WP), lambda i: (i, 0)),
            pl.BlockSpec((r, 16), lambda i: (i, 0)),
        ],
        out_shape=[
            jax.ShapeDtypeStruct((N, WP), jnp.float32),
            jax.ShapeDtypeStruct((N, 16), jnp.float32),
        ],
    )(acc2d, acc2d, xwp, aux, bv, w, asv, adv)


@functools.lru_cache(maxsize=None)
def _mesh():
    return plsc.VectorSubcoreMesh(
        core_axis_name="c", subcore_axis_name="s",
        num_cores=NC, num_subcores=NS)


def _sc_edge(xwp, aux, src_idx, dst_idx):
    e = src_idx.shape[0]
    ept = e // NW
    chunk = 80
    nchunk = ept // chunk
    ngrp = chunk // L
    rows_per_tile = N // NS

    @functools.partial(
        pl.kernel,
        out_type=jax.ShapeDtypeStruct((NC, N, WP), jnp.float32),
        mesh=_mesh(),
        compiler_params=pltpu.CompilerParams(
            use_tc_tiling_on_sc=False, needs_layout_passes=False),
        scratch_types=[
            pltpu.VMEM((6, chunk), jnp.int32),
            pltpu.VMEM((6, chunk), jnp.int32),
            pltpu.VMEM((3, chunk, 16), jnp.float32),
            pltpu.VMEM((3, chunk, WP), jnp.float32),
            pltpu.VMEM_SHARED((N, WP), jnp.float32),
            pltpu.SemaphoreType.DMA,
            pltpu.SemaphoreType.DMA,
            pltpu.SemaphoreType.DMA,
        ],
    )
    def edge_kernel(xwp_hbm, aux_hbm, si_hbm, di_hbm, out_hbm,
                    srcc_v, dstc_v, auxd_v, rows_v, acc_sh,
                    isem, gsem, ssem):
        cid = lax.axis_index("c")
        sid = lax.axis_index("s")
        wid = cid * NS + sid
        eb = wid * ept

        z16 = jnp.zeros((L,), jnp.float32)

        @pl.loop(0, chunk)
        def _(r):
            for q in range(WP // L):
                rows_v[0, r, pl.ds(q * L, L)] = z16

        row0 = sid * rows_per_tile
        nfull, rem = rows_per_tile // chunk, rows_per_tile % chunk
        for i in range(nfull):
            pltpu.sync_copy(rows_v.at[0],
                            acc_sh.at[pl.ds(row0 + i * chunk, chunk)])
        if rem:
            pltpu.sync_copy(rows_v.at[0, pl.ds(0, rem)],
                            acc_sh.at[pl.ds(row0 + nfull * chunk, rem)])
        plsc.subcore_barrier()

        one16 = jnp.ones((L,), jnp.int32)
        iota16 = jax.lax.iota(jnp.int32, L)

        def stage_idx(c, s6):
            base = eb + c * chunk
            pltpu.async_copy(si_hbm.at[pl.ds(base, chunk)],
                             srcc_v.at[s6], isem)
            pltpu.async_copy(di_hbm.at[pl.ds(base, chunk)],
                             dstc_v.at[s6], isem)

        def wait_idx(c, s6):
            base = eb + c * chunk
            pltpu.make_async_copy(si_hbm.at[pl.ds(base, chunk)],
                                  srcc_v.at[s6], isem).wait()
            pltpu.make_async_copy(di_hbm.at[pl.ds(base, chunk)],
                                  dstc_v.at[s6], isem).wait()

        def issue_gathers(s6, s3):
            pltpu.async_copy(xwp_hbm.at[srcc_v.at[s6]], rows_v.at[s3], gsem)
            pltpu.async_copy(aux_hbm.at[dstc_v.at[s6]], auxd_v.at[s3], gsem)

        def wait_gathers(s6, s3):
            pltpu.make_async_copy(
                xwp_hbm.at[srcc_v.at[s6]], rows_v.at[s3], gsem).wait()
            pltpu.make_async_copy(
                aux_hbm.at[srcc_v.at[s6]], auxd_v.at[s3], gsem).wait()

        def wait_scatter(s6, s3):
            pltpu.make_async_copy(
                rows_v.at[s3], acc_sh.at[dstc_v.at[s6]], ssem).wait()

        def compute(s):
            sv = jnp.full((L,), s, jnp.int32)

            @pl.loop(0, ngrp)
            def _(g):
                idx16 = iota16 + g * L
                a_s = plsc.load_gather(
                    rows_v, [sv, idx16, jnp.full((L,), D + 1, jnp.int32)])
                a_d = plsc.load_gather(auxd_v, [sv, idx16, one16])
                ez = a_s + a_d
                ez = jnp.maximum(ez, 0.2 * ez)
                ee = jnp.exp(ez)
                for j in range(L):
                    ev = lax.gather(
                        ee, jnp.full((L, 1), j, jnp.int32),
                        lax.GatherDimensionNumbers(
                            offset_dims=(), collapsed_slice_dims=(0,),
                            start_index_map=(0,)),
                        (1,), mode=lax.GatherScatterMode.PROMISE_IN_BOUNDS)
                    r = g * L + j
                    for q in range(D // L):
                        sl = pl.ds(q * L, L)
                        rows_v[s, r, sl] = rows_v[s, r, sl] * ev
                    rows_v[s, r, pl.ds(D, L)] = ev

        def scatter(s6, s3):
            pltpu.async_copy(rows_v.at[s3], acc_sh.at[dstc_v.at[s6]], ssem,
                             add=True)

        def sub_iter(c, s6, prefetch, stage4, may_be_first):
            s3 = s6 % 3
            wait_gathers(s6, s3)
            compute(s3)
            if prefetch:
                sp3 = (s3 + 2) % 3
                sp6 = (s6 + 2) % 6
                w6 = (s6 + 5) % 6
                if may_be_first:
                    @pl.when(c >= 1)
                    def _():
                        wait_scatter(w6, sp3)
                else:
                    wait_scatter(w6, sp3)
                wait_idx(c + 2, sp6)
                issue_gathers(sp6, sp3)
            if stage4:
                stage_idx(c + 4, (s6 + 4) % 6)
            scatter(s6, s3)

        for k in range(4):
            stage_idx(k, k)
        wait_idx(0, 0)
        issue_gathers(0, 0)
        wait_idx(1, 1)
        issue_gathers(1, 1)

        @pl.loop(0, nchunk // 6)
        def _(t):
            c0 = t * 6
            for k in range(6):
                sub_iter(c0 + k, k, True, True, k == 0 and True)

        nmain = (nchunk // 6) * 6
        for k in range(nchunk - nmain):
            c = nmain + k
            sub_iter(c, k, c + 2 < nchunk, c + 4 < nchunk, False)

        for c in range(max(nchunk - 3, 0), nchunk):
            wait_scatter(c % 6, c % 3)

        plsc.subcore_barrier()
        pltpu.sync_copy(acc_sh.at[pl.ds(row0, rows_per_tile)],
                        out_hbm.at[cid, pl.ds(row0, rows_per_tile)])

    return edge_kernel(xwp, aux, src_idx, dst_idx)


def _row8(v):
    return jnp.zeros((8, D), jnp.float32).at[0].set(v.reshape(-1))


def kernel(x, edge_index, W1, a_src1, a_dst1, b1, W2, a_src2, a_dst2, b2):
    asv1, adv1 = _row8(a_src1), _row8(a_dst1)
    asv2, adv2 = _row8(a_src2), _row8(a_dst2)
    bv1, bv2 = _row8(b1), _row8(b2)

    src_idx, dst_idx = edge_index[0], edge_index[1]

    xwp1, aux1 = _tc_pre(x, W1, asv1, adv1)
    acc = _sc_edge(xwp1, aux1, src_idx, dst_idx).reshape(2 * N, WP)
    xwp2, aux2 = _tc_postpre(acc, xwp1, aux1, bv1, W2, asv2, adv2)
    acc2 = _sc_edge(xwp2, aux2, src_idx, dst_idx).reshape(2 * N, WP)
    return _tc_post(acc2, xwp2, aux2, bv2)
